# Initial kernel scaffold; baseline (speedup 1.0000x reference)
import jax
import jax.numpy as jnp
from jax.experimental import pallas as pl


def kernel(tokens, table):
    raise NotImplementedError("write your pallas kernel here")


# speedup vs baseline: 2.3601x; 2.3601x over previous
import functools
import math

import jax
import jax.numpy as jnp
from jax import lax
from jax.experimental import pallas as pl
from jax.experimental.pallas import tpu as pltpu
from jax.experimental.pallas import tpu_sc as plsc

VOCAB = 100000
EMB = 128
SCALE = math.sqrt(EMB)

NC = 2
NS = 16
NW = NC * NS

CH = 128
NBUF = 2


def _scale_body(t_ref, o_ref):
    o_ref[...] = t_ref[...] * SCALE


def _scale_table(table):
    v, d = table.shape
    blk = 1000
    return pl.pallas_call(
        _scale_body,
        out_shape=jax.ShapeDtypeStruct((v, d), jnp.float32),
        grid=(v // blk,),
        in_specs=[pl.BlockSpec((blk, d), lambda i: (i, 0))],
        out_specs=pl.BlockSpec((blk, d), lambda i: (i, 0)),
    )(table)


def _make_gather(n_total):
    assert n_total % (NW * CH) == 0
    b_per_w = n_total // NW
    nchunk = b_per_w // CH
    mesh = plsc.VectorSubcoreMesh(core_axis_name="c", subcore_axis_name="s")

    @functools.partial(
        pl.kernel,
        out_type=jax.ShapeDtypeStruct((n_total, EMB), jnp.float32),
        mesh=mesh,
        scratch_types=(
            [pltpu.VMEM((nchunk, CH), jnp.int32)]
            + [pltpu.VMEM((CH, EMB), jnp.float32) for _ in range(NBUF)]
            + [pltpu.SemaphoreType.DMA for _ in range(2 * NBUF)]
        ),
    )
    def gather(tok_hbm, table_hbm, out_hbm, idx_v, *rest):
        bufs = rest[:NBUF]
        gsems = rest[NBUF:2 * NBUF]
        osems = rest[2 * NBUF:]
        wid = lax.axis_index("s") * NC + lax.axis_index("c")
        base = wid * b_per_w

        pltpu.sync_copy(tok_hbm.at[wid], idx_v)

        def start_g(j, b):
            return pltpu.async_copy(table_hbm.at[idx_v.at[j]], bufs[b],
                                    gsems[b])

        def start_o(j, b):
            return pltpu.async_copy(
                bufs[b], out_hbm.at[pl.ds(base + j * CH, CH)], osems[b])

        g_cp = [None] * NBUF
        o_cp = [None] * NBUF
        g_cp[0] = start_g(0, 0)
        for j in range(nchunk):
            b = j % NBUF
            nb = (j + 1) % NBUF
            if j + 1 < nchunk:
                if j + 1 >= NBUF:
                    o_cp[nb].wait()
                g_cp[nb] = start_g(j + 1, nb)
            g_cp[b].wait()
            o_cp[b] = start_o(j, b)
        for b in range(NBUF):
            if o_cp[b] is not None:
                o_cp[b].wait()

    return gather


def kernel(tokens, table):
    bsz, seq = tokens.shape
    n_total = bsz * seq
    tok = tokens.astype(jnp.int32).reshape(NW, n_total // (NW * CH), CH)
    table_scaled = _scale_table(table)
    out = _make_gather(n_total)(tok, table_scaled)
    return out.reshape(bsz, seq, EMB)


# speedup vs baseline: 3.5906x; 1.5214x over previous
import functools
import math

import jax
import jax.numpy as jnp
from jax import lax
from jax.experimental import pallas as pl
from jax.experimental.pallas import tpu as pltpu
from jax.experimental.pallas import tpu_sc as plsc

VOCAB = 100000
EMB = 128
SCALE = math.sqrt(EMB)

NC = 2
NS = 16
NW = NC * NS

NBUF = 4


def _scale_body(t_ref, o_ref):
    o_ref[...] = t_ref[...] * SCALE


def _scale_table(table):
    v, d = table.shape
    blk = 1000
    return pl.pallas_call(
        _scale_body,
        out_shape=jax.ShapeDtypeStruct((v, d), jnp.float32),
        grid=(v // blk,),
        in_specs=[pl.BlockSpec((blk, d), lambda i: (i, 0))],
        out_specs=pl.BlockSpec((blk, d), lambda i: (i, 0)),
    )(table)


def _make_gather(bsz, seq):
    assert bsz % NW == 0
    b_per_w = bsz // NW
    mesh = plsc.VectorSubcoreMesh(core_axis_name="c", subcore_axis_name="s")

    @functools.partial(
        pl.kernel,
        out_type=jax.ShapeDtypeStruct((bsz, seq, EMB), jnp.float32),
        mesh=mesh,
        scratch_types=(
            [pltpu.VMEM((b_per_w, seq), jnp.int32)]
            + [pltpu.VMEM((seq, EMB), jnp.float32) for _ in range(NBUF)]
            + [pltpu.SemaphoreType.DMA for _ in range(2 * NBUF)]
        ),
    )
    def gather(tok_hbm, table_hbm, out_hbm, idx_v, *rest):
        bufs = rest[:NBUF]
        gsems = rest[NBUF:2 * NBUF]
        osems = rest[2 * NBUF:]
        wid = lax.axis_index("s") * NC + lax.axis_index("c")
        base = wid * b_per_w

        pltpu.sync_copy(tok_hbm.at[wid], idx_v)

        def start_g(j, b):
            return pltpu.async_copy(table_hbm.at[idx_v.at[j]], bufs[b],
                                    gsems[b])

        def start_o(j, b):
            return pltpu.async_copy(bufs[b], out_hbm.at[base + j], osems[b])

        g_cp = [None] * NBUF
        o_cp = [None] * NBUF
        for j in range(NBUF):
            g_cp[j] = start_g(j, j)
        for j in range(b_per_w):
            b = j % NBUF
            m = j + NBUF // 2
            if NBUF <= m < b_per_w:
                s = m % NBUF
                o_cp[s].wait()
                g_cp[s] = start_g(m, s)
            g_cp[b].wait()
            o_cp[b] = start_o(j, b)
        for j in range(b_per_w - NBUF, b_per_w):
            o_cp[j % NBUF].wait()

    return gather


def kernel(tokens, table):
    bsz, seq = tokens.shape
    tok = tokens.astype(jnp.int32).reshape(NW, bsz // NW, seq)
    table_scaled = _scale_table(table)
    return _make_gather(bsz, seq)(tok, table_scaled)
